# baseline (speedup 1.0000x reference)
import functools

import jax
import jax.numpy as jnp
from jax import lax
from jax.experimental import pallas as pl
from jax.experimental.pallas import tpu as pltpu
from jax.experimental.pallas import tpu_sc as plsc

NC = 2
NS = 16
CH = 128


def _edge_embed_body(ea_ref, w_ref, b_ref, o_ref):
    o_ref[...] = (
        jnp.dot(ea_ref[...], w_ref[...], preferred_element_type=jnp.float32)
        + b_ref[...]
    )


def _edge_embed(ea, w, b, blk=8000):
    E, DE = ea.shape
    D = w.shape[1]
    return pl.pallas_call(
        _edge_embed_body,
        grid=(E // blk,),
        in_specs=[
            pl.BlockSpec((blk, DE), lambda i: (i, 0)),
            pl.BlockSpec((DE, D), lambda i: (0, 0)),
            pl.BlockSpec((1, D), lambda i: (0, 0)),
        ],
        out_specs=pl.BlockSpec((blk, D), lambda i: (i, 0)),
        out_shape=jax.ShapeDtypeStruct((E, D), jnp.float32),
    )(ea, w, b.reshape(1, D))


@functools.lru_cache(maxsize=None)
def _make_sc_edge_kernel(N, D, E, NCH):
    EPW = E // NS
    Dh = D // NC
    RPT = ((N + NS - 1) // NS + 7) // 8 * 8
    LASTR = N - RPT * (NS - 1)
    NLSL = Dh // 16
    ACC_N = N + 8

    mesh = plsc.VectorSubcoreMesh(core_axis_name="c", subcore_axis_name="s")

    @functools.partial(
        pl.kernel,
        out_type=jax.ShapeDtypeStruct((NC, N, Dh), jnp.float32),
        mesh=mesh,
        compiler_params=pltpu.CompilerParams(use_tc_tiling_on_sc=False),
        scratch_types=[
            pltpu.VMEM((NCH, CH), jnp.int32),
            pltpu.VMEM((NCH, CH), jnp.int32),
            pltpu.VMEM((CH, D), jnp.float32),
            pltpu.VMEM((CH, D), jnp.float32),
            pltpu.VMEM((CH, Dh), jnp.float32),
            pltpu.VMEM_SHARED((ACC_N, Dh), jnp.float32),
            pltpu.SemaphoreType.DMA,
            pltpu.SemaphoreType.DMA,
        ],
    )
    def k(x_hbm, e_hbm, srcm_hbm, dstm_hbm, z_hbm, out_hbm,
          sidx, didx, xb, eb, mbuf, acc, gsem, esem):
        c = lax.axis_index("c")
        s = lax.axis_index("s")
        cbase = c * Dh

        @pl.when(s < NS - 1)
        def _():
            pltpu.sync_copy(z_hbm.at[pl.ds(s * RPT, RPT)],
                            acc.at[pl.ds(s * RPT, RPT)])

        @pl.when(s == NS - 1)
        def _():
            pltpu.sync_copy(z_hbm.at[pl.ds((NS - 1) * RPT, LASTR)],
                            acc.at[pl.ds((NS - 1) * RPT, LASTR)])

        pltpu.sync_copy(srcm_hbm.at[s], sidx)
        pltpu.sync_copy(dstm_hbm.at[s], didx)
        plsc.subcore_barrier()

        @pl.loop(0, NCH)
        def _chunk(j):
            eoff = s * EPW + jnp.minimum(j * CH, EPW - CH)
            gcp = pltpu.async_copy(x_hbm.at[sidx.at[j]], xb, gsem)
            ecp = pltpu.async_copy(e_hbm.at[pl.ds(eoff, CH)], eb, esem)
            gcp.wait()
            ecp.wait()

            @pl.loop(0, CH)
            def _row(r):
                for t in range(NLSL):
                    src_sl = pl.ds(cbase + t * 16, 16)
                    dst_sl = pl.ds(t * 16, 16)
                    mbuf[r, dst_sl] = jnp.maximum(
                        xb[r, src_sl] + eb[r, src_sl], 0.0)

            pltpu.sync_copy(mbuf, acc.at[didx.at[j]], add=True)

        plsc.subcore_barrier()

        @pl.when(s < NS - 1)
        def _():
            pltpu.sync_copy(acc.at[pl.ds(s * RPT, RPT)],
                            out_hbm.at[c, pl.ds(s * RPT, RPT)])

        @pl.when(s == NS - 1)
        def _():
            pltpu.sync_copy(acc.at[pl.ds((NS - 1) * RPT, LASTR)],
                            out_hbm.at[c, pl.ds((NS - 1) * RPT, LASTR)])

    return k


def _sc_edge_stage(xh, e, src_m, dst_m, zeros_acc):
    N, D = xh.shape
    k = _make_sc_edge_kernel(N, D, e.shape[0], src_m.shape[1])
    return k(xh, e, src_m, dst_m, zeros_acc)


def _bn(h, g, b):
    mu = jnp.mean(h, axis=0, keepdims=True)
    var = jnp.mean((h - mu) ** 2, axis=0, keepdims=True)
    return (h - mu) * lax.rsqrt(var + 1e-5) * g + b


def _leaky(h):
    return jnp.where(h >= 0.0, h, 0.01 * h)


def _node0_body(x_ref, p_ref, w1_ref, b1_ref, g1_ref, be1_ref, w2_ref, b2_ref,
                og_ref, ob_ref, eps_ref, o_ref):
    aggr = jnp.concatenate([p_ref[0], p_ref[1]], axis=1)
    z = (1.0 + eps_ref[0, 0]) * x_ref[...] + aggr
    h = jnp.dot(z, w1_ref[...], preferred_element_type=jnp.float32) + b1_ref[...]
    h = _leaky(_bn(h, g1_ref[...], be1_ref[...]))
    h = jnp.dot(h, w2_ref[...], preferred_element_type=jnp.float32) + b2_ref[...]
    h = _leaky(_bn(h, og_ref[...], ob_ref[...]))
    o_ref[...] = h


def _node0(x, parts, w1, b1, g1, be1, w2, b2, og, ob, eps):
    N, D = x.shape
    H = w1.shape[1]
    r = lambda v: v.reshape(1, -1)
    return pl.pallas_call(
        _node0_body,
        out_shape=jax.ShapeDtypeStruct((N, H), jnp.float32),
    )(x, parts, w1, r(b1), r(g1), r(be1), w2, r(b2), r(og), r(ob),
      eps.reshape(1, 1))


def _node1_body(x_ref, p_ref, w1_ref, b1_ref, g1_ref, be1_ref, w2_ref, b2_ref,
                wf_ref, bf_ref, eps_ref, o_ref):
    aggr = jnp.concatenate([p_ref[0], p_ref[1]], axis=1)
    z = (1.0 + eps_ref[0, 0]) * x_ref[...] + aggr
    h = jnp.dot(z, w1_ref[...], preferred_element_type=jnp.float32) + b1_ref[...]
    h = _leaky(_bn(h, g1_ref[...], be1_ref[...]))
    h = jnp.dot(h, w2_ref[...], preferred_element_type=jnp.float32) + b2_ref[...]
    logits = (jnp.dot(h, wf_ref[...], preferred_element_type=jnp.float32)
              + bf_ref[...])
    m = jnp.max(logits, axis=1, keepdims=True)
    ex = jnp.exp(logits - m)
    o_ref[...] = ex / jnp.sum(ex, axis=1, keepdims=True)


def _node1(h, parts, w1, b1, g1, be1, w2, b2, wf_pad, bf_pad, eps):
    N = h.shape[0]
    r = lambda v: v.reshape(1, -1)
    return pl.pallas_call(
        _node1_body,
        out_shape=jax.ShapeDtypeStruct((N, wf_pad.shape[1]), jnp.float32),
    )(h, parts, w1, r(b1), r(g1), r(be1), w2, r(b2), wf_pad, r(bf_pad),
      eps.reshape(1, 1))


def kernel(x, edge_index, edge_attr, le0_W, le0_b, eps0, W1_0, b1_0, g1_0,
           be1_0, W2_0, b2_0, og_0, ob_0, le1_W, le1_b, eps1, W1_1, b1_1,
           g1_1, be1_1, W2_1, b2_1, og_1, ob_1, Wf, bf):
    N, D = x.shape
    E = edge_index.shape[1]
    C = Wf.shape[1]
    Dh = D // NC

    src = edge_index[0]
    dst = edge_index[1]
    EPW = E // NS
    NCF = EPW // CH
    NCH = NCF + (1 if EPW % CH else 0)
    srcw = src.reshape(NS, EPW)
    dstw = dst.reshape(NS, EPW)
    parts_s = [srcw[:, :NCF * CH].reshape(NS, NCF, CH)]
    parts_d = [dstw[:, :NCF * CH].reshape(NS, NCF, CH)]
    if EPW % CH:
        dup = NCH * CH - EPW
        parts_s.append(srcw[:, EPW - CH:].reshape(NS, 1, CH))
        dst_last = jnp.where(jnp.arange(CH) < dup, N, dstw[:, EPW - CH:])
        parts_d.append(dst_last.reshape(NS, 1, CH))
    src_m = jnp.concatenate(parts_s, axis=1) if len(parts_s) > 1 else parts_s[0]
    dst_m = jnp.concatenate(parts_d, axis=1) if len(parts_d) > 1 else parts_d[0]

    zeros_acc = jnp.zeros((N, Dh), jnp.float32)

    wf_pad = jnp.zeros((Wf.shape[0], 128), jnp.float32).at[:, :C].set(Wf)
    bf_pad = jnp.full((128,), -1e30, jnp.float32).at[:C].set(bf)

    e0 = _edge_embed(edge_attr, le0_W, le0_b)
    e1 = _edge_embed(edge_attr, le1_W, le1_b)

    parts0 = _sc_edge_stage(x, e0, src_m, dst_m, zeros_acc)
    h = _node0(x, parts0, W1_0, b1_0, g1_0, be1_0, W2_0, b2_0, og_0, ob_0,
               eps0)
    parts1 = _sc_edge_stage(h, e1, src_m, dst_m, zeros_acc)
    probs = _node1(h, parts1, W1_1, b1_1, g1_1, be1_1, W2_1, b2_1, wf_pad,
                   bf_pad, eps1)
    return probs[:, :C]


# speedup vs baseline: 1.9024x; 1.9024x over previous
import functools

import jax
import jax.numpy as jnp
from jax import lax
from jax.experimental import pallas as pl
from jax.experimental.pallas import tpu as pltpu
from jax.experimental.pallas import tpu_sc as plsc

NC = 2
NS = 16
CH = 128


def _edge_embed_body(ea_ref, w_ref, b_ref, o_ref):
    for c in range(NC):
        o_ref[c] = (
            jnp.dot(ea_ref[...], w_ref[c], preferred_element_type=jnp.float32)
            + b_ref[c]
        )


def _edge_embed(ea2, wpk, bpk, blk=4000):
    Eh, DE2 = ea2.shape
    return pl.pallas_call(
        _edge_embed_body,
        grid=(Eh // blk,),
        in_specs=[
            pl.BlockSpec((blk, DE2), lambda i: (i, 0)),
            pl.BlockSpec((NC, DE2, 128), lambda i: (0, 0, 0)),
            pl.BlockSpec((NC, 1, 128), lambda i: (0, 0, 0)),
        ],
        out_specs=pl.BlockSpec((NC, blk, 128), lambda i: (0, i, 0)),
        out_shape=jax.ShapeDtypeStruct((NC, Eh, 128), jnp.float32),
    )(ea2, wpk, bpk)


@functools.lru_cache(maxsize=None)
def _make_sc_edge_kernel(N, D, E, NCH):
    EPW = E // NS
    Dh = D // NC
    RPT = ((N + NS - 1) // NS + 7) // 8 * 8
    LASTR = N - RPT * (NS - 1)
    NLSL = Dh // 16
    ACC_N = N + 8

    mesh = plsc.VectorSubcoreMesh(core_axis_name="c", subcore_axis_name="s")

    @functools.partial(
        pl.kernel,
        out_type=jax.ShapeDtypeStruct((NC, N, Dh), jnp.float32),
        mesh=mesh,
        compiler_params=pltpu.CompilerParams(use_tc_tiling_on_sc=False),
        scratch_types=[
            pltpu.VMEM((NCH, CH), jnp.int32),
            pltpu.VMEM((NCH, CH), jnp.int32),
            pltpu.VMEM((CH, D // NC), jnp.float32),
            pltpu.VMEM((CH // 2, D), jnp.float32),
            pltpu.VMEM((CH, D // NC), jnp.float32),
            pltpu.VMEM_SHARED((ACC_N, D // NC), jnp.float32),
            pltpu.SemaphoreType.DMA,
            pltpu.SemaphoreType.DMA,
        ],
    )
    def k(x_hbm, e_hbm, srcm_hbm, dstm_hbm, z_hbm, out_hbm,
          sidx, didx, xb, eb, mbuf, acc, gsem, esem):
        c = lax.axis_index("c")
        s = lax.axis_index("s")

        @pl.when(s < NS - 1)
        def _():
            pltpu.sync_copy(z_hbm.at[pl.ds(s * RPT, RPT)],
                            acc.at[pl.ds(s * RPT, RPT)])

        @pl.when(s == NS - 1)
        def _():
            pltpu.sync_copy(z_hbm.at[pl.ds((NS - 1) * RPT, LASTR)],
                            acc.at[pl.ds((NS - 1) * RPT, LASTR)])

        pltpu.sync_copy(srcm_hbm.at[s], sidx)
        pltpu.sync_copy(dstm_hbm.at[s], didx)
        plsc.subcore_barrier()

        @pl.loop(0, NCH)
        def _chunk(j):
            eoff = s * EPW + jnp.minimum(j * CH, EPW - CH)
            gcp = pltpu.async_copy(x_hbm.at[c].at[sidx.at[j]], xb, gsem)
            ecp = pltpu.async_copy(
                e_hbm.at[c].at[pl.ds(lax.div(eoff, 2), CH // 2)], eb, esem)
            gcp.wait()
            ecp.wait()

            @pl.loop(0, CH, step=2)
            def _pair(r):
                rh = lax.div(r, 2)
                for t in range(NLSL):
                    sl = pl.ds(t * 16, 16)
                    mbuf[r, sl] = jnp.maximum(
                        xb[r, sl] + eb[rh, pl.ds(t * 16, 16)], 0.0)
                    mbuf[r + 1, sl] = jnp.maximum(
                        xb[r + 1, sl] + eb[rh, pl.ds(Dh + t * 16, 16)], 0.0)

            pltpu.sync_copy(mbuf, acc.at[didx.at[j]], add=True)

        plsc.subcore_barrier()

        @pl.when(s < NS - 1)
        def _():
            pltpu.sync_copy(acc.at[pl.ds(s * RPT, RPT)],
                            out_hbm.at[c, pl.ds(s * RPT, RPT)])

        @pl.when(s == NS - 1)
        def _():
            pltpu.sync_copy(acc.at[pl.ds((NS - 1) * RPT, LASTR)],
                            out_hbm.at[c, pl.ds((NS - 1) * RPT, LASTR)])

    return k


def _sc_edge_stage(x2, e2p, src_m, dst_m, zeros_acc):
    _, N, Dh = x2.shape
    k = _make_sc_edge_kernel(N, Dh * NC, 2 * e2p.shape[1], src_m.shape[1])
    return k(x2, e2p, src_m, dst_m, zeros_acc)


def _bn(h, g, b):
    mu = jnp.mean(h, axis=0, keepdims=True)
    var = jnp.mean((h - mu) ** 2, axis=0, keepdims=True)
    return (h - mu) * lax.rsqrt(var + 1e-5) * g + b


def _leaky(h):
    return jnp.where(h >= 0.0, h, 0.01 * h)


def _node0_body(x_ref, p_ref, w1_ref, b1_ref, g1_ref, be1_ref, w2_ref, b2_ref,
                og_ref, ob_ref, eps_ref, o_ref):
    dh = o_ref.shape[2]
    aggr = jnp.concatenate([p_ref[0], p_ref[1]], axis=1)
    z = (1.0 + eps_ref[0, 0]) * x_ref[...] + aggr
    h = jnp.dot(z, w1_ref[...], preferred_element_type=jnp.float32) + b1_ref[...]
    h = _leaky(_bn(h, g1_ref[...], be1_ref[...]))
    h = jnp.dot(h, w2_ref[...], preferred_element_type=jnp.float32) + b2_ref[...]
    h = _leaky(_bn(h, og_ref[...], ob_ref[...]))
    o_ref[0] = h[:, :dh]
    o_ref[1] = h[:, dh:]


def _node0(x, parts, w1, b1, g1, be1, w2, b2, og, ob, eps):
    N, D = x.shape
    H = w1.shape[1]
    r = lambda v: v.reshape(1, -1)
    return pl.pallas_call(
        _node0_body,
        out_shape=jax.ShapeDtypeStruct((NC, N, H // NC), jnp.float32),
    )(x, parts, w1, r(b1), r(g1), r(be1), w2, r(b2), r(og), r(ob),
      eps.reshape(1, 1))


def _node1_body(x_ref, p_ref, w1_ref, b1_ref, g1_ref, be1_ref, w2_ref, b2_ref,
                wf_ref, bf_ref, eps_ref, o_ref):
    x = jnp.concatenate([x_ref[0], x_ref[1]], axis=1)
    aggr = jnp.concatenate([p_ref[0], p_ref[1]], axis=1)
    z = (1.0 + eps_ref[0, 0]) * x + aggr
    h = jnp.dot(z, w1_ref[...], preferred_element_type=jnp.float32) + b1_ref[...]
    h = _leaky(_bn(h, g1_ref[...], be1_ref[...]))
    h = jnp.dot(h, w2_ref[...], preferred_element_type=jnp.float32) + b2_ref[...]
    logits = (jnp.dot(h, wf_ref[...], preferred_element_type=jnp.float32)
              + bf_ref[...])
    m = jnp.max(logits, axis=1, keepdims=True)
    ex = jnp.exp(logits - m)
    o_ref[...] = ex / jnp.sum(ex, axis=1, keepdims=True)


def _node1(h2, parts, w1, b1, g1, be1, w2, b2, wf_pad, bf_pad, eps):
    N = h2.shape[1]
    r = lambda v: v.reshape(1, -1)
    return pl.pallas_call(
        _node1_body,
        out_shape=jax.ShapeDtypeStruct((N, wf_pad.shape[1]), jnp.float32),
    )(h2, parts, w1, r(b1), r(g1), r(be1), w2, r(b2), wf_pad, r(bf_pad),
      eps.reshape(1, 1))


def kernel(x, edge_index, edge_attr, le0_W, le0_b, eps0, W1_0, b1_0, g1_0,
           be1_0, W2_0, b2_0, og_0, ob_0, le1_W, le1_b, eps1, W1_1, b1_1,
           g1_1, be1_1, W2_1, b2_1, og_1, ob_1, Wf, bf):
    N, D = x.shape
    E = edge_index.shape[1]
    C = Wf.shape[1]
    Dh = D // NC

    src = edge_index[0]
    dst = edge_index[1]
    EPW = E // NS
    NCF = EPW // CH
    NCH = NCF + (1 if EPW % CH else 0)
    srcw = src.reshape(NS, EPW)
    dstw = dst.reshape(NS, EPW)
    parts_s = [srcw[:, :NCF * CH].reshape(NS, NCF, CH)]
    parts_d = [dstw[:, :NCF * CH].reshape(NS, NCF, CH)]
    if EPW % CH:
        dup = NCH * CH - EPW
        parts_s.append(srcw[:, EPW - CH:].reshape(NS, 1, CH))
        dst_last = jnp.where(jnp.arange(CH) < dup, N, dstw[:, EPW - CH:])
        parts_d.append(dst_last.reshape(NS, 1, CH))
    src_m = jnp.concatenate(parts_s, axis=1) if len(parts_s) > 1 else parts_s[0]
    dst_m = jnp.concatenate(parts_d, axis=1) if len(parts_d) > 1 else parts_d[0]

    zeros_acc = jnp.zeros((N, Dh), jnp.float32)
    x2 = jnp.stack([x[:, :Dh], x[:, Dh:]])

    wf_pad = jnp.zeros((Wf.shape[0], 128), jnp.float32).at[:, :C].set(Wf)
    bf_pad = jnp.full((128,), -1e30, jnp.float32).at[:C].set(bf)

    DE = edge_attr.shape[1]
    ea2 = edge_attr.reshape(E // 2, 2 * DE)

    def packw(w, b):
        wpk = jnp.zeros((NC, 2 * DE, 128), jnp.float32)
        bpk = jnp.zeros((NC, 1, 128), jnp.float32)
        for c in range(NC):
            wh = w[:, c * Dh:(c + 1) * Dh]
            wpk = wpk.at[c, :DE, :Dh].set(wh).at[c, DE:, Dh:].set(wh)
            bh = b[c * Dh:(c + 1) * Dh]
            bpk = bpk.at[c, 0, :Dh].set(bh).at[c, 0, Dh:].set(bh)
        return wpk, bpk

    wpk0, bpk0 = packw(le0_W, le0_b)
    wpk1, bpk1 = packw(le1_W, le1_b)
    e0 = _edge_embed(ea2, wpk0, bpk0)
    e1 = _edge_embed(ea2, wpk1, bpk1)

    parts0 = _sc_edge_stage(x2, e0, src_m, dst_m, zeros_acc)
    h2 = _node0(x, parts0, W1_0, b1_0, g1_0, be1_0, W2_0, b2_0, og_0, ob_0,
                eps0)
    parts1 = _sc_edge_stage(h2, e1, src_m, dst_m, zeros_acc)
    probs = _node1(h2, parts1, W1_1, b1_1, g1_1, be1_1, W2_1, b2_1, wf_pad,
                   bf_pad, eps1)
    return probs[:, :C]


# speedup vs baseline: 2.6916x; 1.4148x over previous
import functools

import jax
import jax.numpy as jnp
from jax import lax
from jax.experimental import pallas as pl
from jax.experimental.pallas import tpu as pltpu
from jax.experimental.pallas import tpu_sc as plsc

NC = 2
NS = 16
CH = 128


def _edge_embed_body(ea_ref, w_ref, b_ref, o_ref):
    for c in range(NC):
        o_ref[c] = (
            jnp.dot(ea_ref[...], w_ref[c], preferred_element_type=jnp.float32)
            + b_ref[c]
        )


def _edge_embed(ea2, wpk, bpk, blk=4000):
    Eh, DE2 = ea2.shape
    return pl.pallas_call(
        _edge_embed_body,
        grid=(Eh // blk,),
        in_specs=[
            pl.BlockSpec((blk, DE2), lambda i: (i, 0)),
            pl.BlockSpec((NC, DE2, 128), lambda i: (0, 0, 0)),
            pl.BlockSpec((NC, 1, 128), lambda i: (0, 0, 0)),
        ],
        out_specs=pl.BlockSpec((NC, blk, 128), lambda i: (0, i, 0)),
        out_shape=jax.ShapeDtypeStruct((NC, Eh, 128), jnp.float32),
    )(ea2, wpk, bpk)


@functools.lru_cache(maxsize=None)
def _make_sc_edge_kernel(N, D, E, NCH):
    EPW = E // NS
    Dh = D // NC
    RPT = ((N + NS - 1) // NS + 7) // 8 * 8
    LASTR = N - RPT * (NS - 1)
    NLSL = Dh // 16
    ACC_N = N + 8

    mesh = plsc.VectorSubcoreMesh(core_axis_name="c", subcore_axis_name="s")

    @functools.partial(
        pl.kernel,
        out_type=jax.ShapeDtypeStruct((NC, N, Dh), jnp.float32),
        mesh=mesh,
        compiler_params=pltpu.CompilerParams(use_tc_tiling_on_sc=False),
        scratch_types=[
            pltpu.VMEM((NCH, CH), jnp.int32),
            pltpu.VMEM((NCH, CH), jnp.int32),
            pltpu.VMEM((CH, D // NC), jnp.float32),
            pltpu.VMEM((CH, D // NC), jnp.float32),
            pltpu.VMEM((CH // 2, D), jnp.float32),
            pltpu.VMEM((CH // 2, D), jnp.float32),
            pltpu.VMEM((CH, D // NC), jnp.float32),
            pltpu.VMEM_SHARED((ACC_N, D // NC), jnp.float32),
            pltpu.SemaphoreType.DMA,
            pltpu.SemaphoreType.DMA,
            pltpu.SemaphoreType.DMA,
            pltpu.SemaphoreType.DMA,
        ],
    )
    def k(x_hbm, e_hbm, srcm_hbm, dstm_hbm, z_hbm, out_hbm,
          sidx, didx, xb0, xb1, eb0, eb1, mbuf, acc, g0, g1, e0sem, e1sem):
        c = lax.axis_index("c")
        s = lax.axis_index("s")

        @pl.when(s < NS - 1)
        def _():
            pltpu.sync_copy(z_hbm.at[pl.ds(s * RPT, RPT)],
                            acc.at[pl.ds(s * RPT, RPT)])

        @pl.when(s == NS - 1)
        def _():
            pltpu.sync_copy(z_hbm.at[pl.ds((NS - 1) * RPT, LASTR)],
                            acc.at[pl.ds((NS - 1) * RPT, LASTR)])

        pltpu.sync_copy(srcm_hbm.at[s], sidx)
        pltpu.sync_copy(dstm_hbm.at[s], didx)
        plsc.subcore_barrier()

        def start_loads(j, xb, eb, gsem, esem):
            eoff = s * EPW + jnp.minimum(j * CH, EPW - CH)
            pltpu.async_copy(x_hbm.at[c].at[sidx.at[j]], xb, gsem)
            pltpu.async_copy(
                e_hbm.at[c].at[pl.ds(lax.div(eoff, 2), CH // 2)], eb, esem)

        def process(j, xb, eb, gsem, esem):
            pltpu.make_async_copy(x_hbm.at[c].at[pl.ds(0, CH)], xb, gsem).wait()
            pltpu.make_async_copy(e_hbm.at[c].at[pl.ds(0, CH // 2)], eb,
                                  esem).wait()

            @pl.loop(0, CH, step=2)
            def _pair(r):
                rh = lax.div(r, 2)
                for t in range(NLSL):
                    sl = pl.ds(t * 16, 16)
                    mbuf[r, sl] = jnp.maximum(
                        xb[r, sl] + eb[rh, pl.ds(t * 16, 16)], 0.0)
                    mbuf[r + 1, sl] = jnp.maximum(
                        xb[r + 1, sl] + eb[rh, pl.ds(Dh + t * 16, 16)], 0.0)

            @pl.when(j + 2 < NCH)
            def _():
                start_loads(j + 2, xb, eb, gsem, esem)

            pltpu.sync_copy(mbuf, acc.at[didx.at[j]], add=True)

        start_loads(0, xb0, eb0, g0, e0sem)
        if NCH > 1:
            start_loads(1, xb1, eb1, g1, e1sem)

        @pl.loop(0, NCH - (NCH % 2), step=2)
        def _chunkpair(j):
            process(j, xb0, eb0, g0, e0sem)
            process(j + 1, xb1, eb1, g1, e1sem)

        if NCH % 2:
            process(NCH - 1, xb0, eb0, g0, e0sem)

        plsc.subcore_barrier()

        @pl.when(s < NS - 1)
        def _():
            pltpu.sync_copy(acc.at[pl.ds(s * RPT, RPT)],
                            out_hbm.at[c, pl.ds(s * RPT, RPT)])

        @pl.when(s == NS - 1)
        def _():
            pltpu.sync_copy(acc.at[pl.ds((NS - 1) * RPT, LASTR)],
                            out_hbm.at[c, pl.ds((NS - 1) * RPT, LASTR)])

    return k


def _sc_edge_stage(x2, e2p, src_m, dst_m, zeros_acc):
    _, N, Dh = x2.shape
    k = _make_sc_edge_kernel(N, Dh * NC, 2 * e2p.shape[1], src_m.shape[1])
    return k(x2, e2p, src_m, dst_m, zeros_acc)


def _bn(h, g, b):
    mu = jnp.mean(h, axis=0, keepdims=True)
    var = jnp.mean((h - mu) ** 2, axis=0, keepdims=True)
    return (h - mu) * lax.rsqrt(var + 1e-5) * g + b


def _leaky(h):
    return jnp.where(h >= 0.0, h, 0.01 * h)


def _node0_body(x_ref, p_ref, w1_ref, b1_ref, g1_ref, be1_ref, w2_ref, b2_ref,
                og_ref, ob_ref, eps_ref, o_ref):
    dh = o_ref.shape[2]
    aggr = jnp.concatenate([p_ref[0], p_ref[1]], axis=1)
    z = (1.0 + eps_ref[0, 0]) * x_ref[...] + aggr
    h = jnp.dot(z, w1_ref[...], preferred_element_type=jnp.float32) + b1_ref[...]
    h = _leaky(_bn(h, g1_ref[...], be1_ref[...]))
    h = jnp.dot(h, w2_ref[...], preferred_element_type=jnp.float32) + b2_ref[...]
    h = _leaky(_bn(h, og_ref[...], ob_ref[...]))
    o_ref[0] = h[:, :dh]
    o_ref[1] = h[:, dh:]


def _node0(x, parts, w1, b1, g1, be1, w2, b2, og, ob, eps):
    N, D = x.shape
    H = w1.shape[1]
    r = lambda v: v.reshape(1, -1)
    return pl.pallas_call(
        _node0_body,
        out_shape=jax.ShapeDtypeStruct((NC, N, H // NC), jnp.float32),
    )(x, parts, w1, r(b1), r(g1), r(be1), w2, r(b2), r(og), r(ob),
      eps.reshape(1, 1))


def _node1_body(x_ref, p_ref, w1_ref, b1_ref, g1_ref, be1_ref, w2_ref, b2_ref,
                wf_ref, bf_ref, eps_ref, o_ref):
    x = jnp.concatenate([x_ref[0], x_ref[1]], axis=1)
    aggr = jnp.concatenate([p_ref[0], p_ref[1]], axis=1)
    z = (1.0 + eps_ref[0, 0]) * x + aggr
    h = jnp.dot(z, w1_ref[...], preferred_element_type=jnp.float32) + b1_ref[...]
    h = _leaky(_bn(h, g1_ref[...], be1_ref[...]))
    h = jnp.dot(h, w2_ref[...], preferred_element_type=jnp.float32) + b2_ref[...]
    logits = (jnp.dot(h, wf_ref[...], preferred_element_type=jnp.float32)
              + bf_ref[...])
    m = jnp.max(logits, axis=1, keepdims=True)
    ex = jnp.exp(logits - m)
    o_ref[...] = ex / jnp.sum(ex, axis=1, keepdims=True)


def _node1(h2, parts, w1, b1, g1, be1, w2, b2, wf_pad, bf_pad, eps):
    N = h2.shape[1]
    r = lambda v: v.reshape(1, -1)
    return pl.pallas_call(
        _node1_body,
        out_shape=jax.ShapeDtypeStruct((N, wf_pad.shape[1]), jnp.float32),
    )(h2, parts, w1, r(b1), r(g1), r(be1), w2, r(b2), wf_pad, r(bf_pad),
      eps.reshape(1, 1))


def kernel(x, edge_index, edge_attr, le0_W, le0_b, eps0, W1_0, b1_0, g1_0,
           be1_0, W2_0, b2_0, og_0, ob_0, le1_W, le1_b, eps1, W1_1, b1_1,
           g1_1, be1_1, W2_1, b2_1, og_1, ob_1, Wf, bf):
    N, D = x.shape
    E = edge_index.shape[1]
    C = Wf.shape[1]
    Dh = D // NC

    src = edge_index[0]
    dst = edge_index[1]
    EPW = E // NS
    NCF = EPW // CH
    NCH = NCF + (1 if EPW % CH else 0)
    srcw = src.reshape(NS, EPW)
    dstw = dst.reshape(NS, EPW)
    parts_s = [srcw[:, :NCF * CH].reshape(NS, NCF, CH)]
    parts_d = [dstw[:, :NCF * CH].reshape(NS, NCF, CH)]
    if EPW % CH:
        dup = NCH * CH - EPW
        parts_s.append(srcw[:, EPW - CH:].reshape(NS, 1, CH))
        dst_last = jnp.where(jnp.arange(CH) < dup, N, dstw[:, EPW - CH:])
        parts_d.append(dst_last.reshape(NS, 1, CH))
    src_m = jnp.concatenate(parts_s, axis=1) if len(parts_s) > 1 else parts_s[0]
    dst_m = jnp.concatenate(parts_d, axis=1) if len(parts_d) > 1 else parts_d[0]

    zeros_acc = jnp.zeros((N, Dh), jnp.float32)
    x2 = jnp.stack([x[:, :Dh], x[:, Dh:]])

    wf_pad = jnp.zeros((Wf.shape[0], 128), jnp.float32).at[:, :C].set(Wf)
    bf_pad = jnp.full((128,), -1e30, jnp.float32).at[:C].set(bf)

    DE = edge_attr.shape[1]
    ea2 = edge_attr.reshape(E // 2, 2 * DE)

    def packw(w, b):
        wpk = jnp.zeros((NC, 2 * DE, 128), jnp.float32)
        bpk = jnp.zeros((NC, 1, 128), jnp.float32)
        for c in range(NC):
            wh = w[:, c * Dh:(c + 1) * Dh]
            wpk = wpk.at[c, :DE, :Dh].set(wh).at[c, DE:, Dh:].set(wh)
            bh = b[c * Dh:(c + 1) * Dh]
            bpk = bpk.at[c, 0, :Dh].set(bh).at[c, 0, Dh:].set(bh)
        return wpk, bpk

    wpk0, bpk0 = packw(le0_W, le0_b)
    wpk1, bpk1 = packw(le1_W, le1_b)
    e0 = _edge_embed(ea2, wpk0, bpk0)
    e1 = _edge_embed(ea2, wpk1, bpk1)

    parts0 = _sc_edge_stage(x2, e0, src_m, dst_m, zeros_acc)
    h2 = _node0(x, parts0, W1_0, b1_0, g1_0, be1_0, W2_0, b2_0, og_0, ob_0,
                eps0)
    parts1 = _sc_edge_stage(h2, e1, src_m, dst_m, zeros_acc)
    probs = _node1(h2, parts1, W1_1, b1_1, g1_1, be1_1, W2_1, b2_1, wf_pad,
                   bf_pad, eps1)
    return probs[:, :C]


# speedup vs baseline: 2.7248x; 1.0123x over previous
import functools

import jax
import jax.numpy as jnp
from jax import lax
from jax.experimental import pallas as pl
from jax.experimental.pallas import tpu as pltpu
from jax.experimental.pallas import tpu_sc as plsc

NC = 2
NS = 16
CH = 128


def _edge_embed_body(ea_ref, w_ref, b_ref, o_ref):
    o_ref[...] = (
        jnp.dot(ea_ref[...], w_ref[...], preferred_element_type=jnp.float32)
        + b_ref[...]
    )


def _edge_embed(ea, w, b, blk=8000):
    E, DE = ea.shape
    D = w.shape[1]
    return pl.pallas_call(
        _edge_embed_body,
        grid=(E // blk,),
        in_specs=[
            pl.BlockSpec((blk, DE), lambda i: (i, 0)),
            pl.BlockSpec((DE, D), lambda i: (0, 0)),
            pl.BlockSpec((1, D), lambda i: (0, 0)),
        ],
        out_specs=pl.BlockSpec((blk, D), lambda i: (i, 0)),
        out_shape=jax.ShapeDtypeStruct((E, D), jnp.float32),
    )(ea, w, b.reshape(1, D))


@functools.lru_cache(maxsize=None)
def _make_sc_edge_kernel(N, D, E, NCH):
    EPW = E // NS
    Dh = D // NC
    RPT = ((N + NS - 1) // NS + 7) // 8 * 8
    LASTR = N - RPT * (NS - 1)
    NLSL = Dh // 16
    ACC_N = N + 8

    mesh = plsc.VectorSubcoreMesh(core_axis_name="c", subcore_axis_name="s")

    @functools.partial(
        pl.kernel,
        out_type=jax.ShapeDtypeStruct((NC, N, Dh), jnp.float32),
        mesh=mesh,
        compiler_params=pltpu.CompilerParams(use_tc_tiling_on_sc=False),
        scratch_types=[
            pltpu.VMEM((NCH, CH), jnp.int32),
            pltpu.VMEM((NCH, CH), jnp.int32),
            pltpu.VMEM((CH, D // NC), jnp.float32),
            pltpu.VMEM((CH, D // NC), jnp.float32),
            pltpu.VMEM((CH, D // NC), jnp.float32),
            pltpu.VMEM((CH, D // NC), jnp.float32),
            pltpu.VMEM((CH, D // NC), jnp.float32),
            pltpu.VMEM_SHARED((ACC_N, D // NC), jnp.float32),
            pltpu.SemaphoreType.DMA,
            pltpu.SemaphoreType.DMA,
            pltpu.SemaphoreType.DMA,
            pltpu.SemaphoreType.DMA,
        ],
    )
    def k(x_hbm, e_hbm, srcm_hbm, dstm_hbm, z_hbm, out_hbm,
          sidx, didx, xb0, xb1, eb0, eb1, mbuf, acc, g0, g1, e0sem, e1sem):
        c = lax.axis_index("c")
        s = lax.axis_index("s")
        cbase = c * Dh

        @pl.when(s < NS - 1)
        def _():
            pltpu.sync_copy(z_hbm.at[pl.ds(s * RPT, RPT)],
                            acc.at[pl.ds(s * RPT, RPT)])

        @pl.when(s == NS - 1)
        def _():
            pltpu.sync_copy(z_hbm.at[pl.ds((NS - 1) * RPT, LASTR)],
                            acc.at[pl.ds((NS - 1) * RPT, LASTR)])

        pltpu.sync_copy(srcm_hbm.at[s], sidx)
        pltpu.sync_copy(dstm_hbm.at[s], didx)
        plsc.subcore_barrier()

        def start_loads(j, xb, eb, gsem, esem):
            eoff = s * EPW + jnp.minimum(j * CH, EPW - CH)
            pltpu.async_copy(x_hbm.at[c].at[sidx.at[j]], xb, gsem)
            pltpu.async_copy(
                e_hbm.at[pl.ds(eoff, CH), pl.ds(cbase, Dh)], eb, esem)

        def process(j, xb, eb, gsem, esem):
            pltpu.make_async_copy(x_hbm.at[c].at[pl.ds(0, CH)], xb, gsem).wait()
            pltpu.make_async_copy(e_hbm.at[pl.ds(0, CH), pl.ds(0, Dh)], eb,
                                  esem).wait()

            @pl.loop(0, CH)
            def _row(r):
                for t in range(NLSL):
                    sl = pl.ds(t * 16, 16)
                    mbuf[r, sl] = jnp.maximum(xb[r, sl] + eb[r, sl], 0.0)

            @pl.when(j + 2 < NCH)
            def _():
                start_loads(j + 2, xb, eb, gsem, esem)

            pltpu.sync_copy(mbuf, acc.at[didx.at[j]], add=True)

        start_loads(0, xb0, eb0, g0, e0sem)
        if NCH > 1:
            start_loads(1, xb1, eb1, g1, e1sem)

        @pl.loop(0, NCH - (NCH % 2), step=2)
        def _chunkpair(j):
            process(j, xb0, eb0, g0, e0sem)
            process(j + 1, xb1, eb1, g1, e1sem)

        if NCH % 2:
            process(NCH - 1, xb0, eb0, g0, e0sem)

        plsc.subcore_barrier()

        @pl.when(s < NS - 1)
        def _():
            pltpu.sync_copy(acc.at[pl.ds(s * RPT, RPT)],
                            out_hbm.at[c, pl.ds(s * RPT, RPT)])

        @pl.when(s == NS - 1)
        def _():
            pltpu.sync_copy(acc.at[pl.ds((NS - 1) * RPT, LASTR)],
                            out_hbm.at[c, pl.ds((NS - 1) * RPT, LASTR)])

    return k


def _sc_edge_stage(x2, e, src_m, dst_m, zeros_acc):
    _, N, Dh = x2.shape
    k = _make_sc_edge_kernel(N, Dh * NC, e.shape[0], src_m.shape[1])
    return k(x2, e, src_m, dst_m, zeros_acc)


def _bn(h, g, b):
    mu = jnp.mean(h, axis=0, keepdims=True)
    var = jnp.mean((h - mu) ** 2, axis=0, keepdims=True)
    return (h - mu) * lax.rsqrt(var + 1e-5) * g + b


def _leaky(h):
    return jnp.where(h >= 0.0, h, 0.01 * h)


def _node0_body(x_ref, p_ref, w1_ref, b1_ref, g1_ref, be1_ref, w2_ref, b2_ref,
                og_ref, ob_ref, eps_ref, o_ref):
    dh = o_ref.shape[2]
    aggr = jnp.concatenate([p_ref[0], p_ref[1]], axis=1)
    z = (1.0 + eps_ref[0, 0]) * x_ref[...] + aggr
    h = jnp.dot(z, w1_ref[...], preferred_element_type=jnp.float32) + b1_ref[...]
    h = _leaky(_bn(h, g1_ref[...], be1_ref[...]))
    h = jnp.dot(h, w2_ref[...], preferred_element_type=jnp.float32) + b2_ref[...]
    h = _leaky(_bn(h, og_ref[...], ob_ref[...]))
    o_ref[0] = h[:, :dh]
    o_ref[1] = h[:, dh:]


def _node0(x, parts, w1, b1, g1, be1, w2, b2, og, ob, eps):
    N, D = x.shape
    H = w1.shape[1]
    r = lambda v: v.reshape(1, -1)
    return pl.pallas_call(
        _node0_body,
        out_shape=jax.ShapeDtypeStruct((NC, N, H // NC), jnp.float32),
    )(x, parts, w1, r(b1), r(g1), r(be1), w2, r(b2), r(og), r(ob),
      eps.reshape(1, 1))


def _node1_body(x_ref, p_ref, w1_ref, b1_ref, g1_ref, be1_ref, w2_ref, b2_ref,
                wf_ref, bf_ref, eps_ref, o_ref):
    x = jnp.concatenate([x_ref[0], x_ref[1]], axis=1)
    aggr = jnp.concatenate([p_ref[0], p_ref[1]], axis=1)
    z = (1.0 + eps_ref[0, 0]) * x + aggr
    h = jnp.dot(z, w1_ref[...], preferred_element_type=jnp.float32) + b1_ref[...]
    h = _leaky(_bn(h, g1_ref[...], be1_ref[...]))
    h = jnp.dot(h, w2_ref[...], preferred_element_type=jnp.float32) + b2_ref[...]
    logits = (jnp.dot(h, wf_ref[...], preferred_element_type=jnp.float32)
              + bf_ref[...])
    m = jnp.max(logits, axis=1, keepdims=True)
    ex = jnp.exp(logits - m)
    o_ref[...] = ex / jnp.sum(ex, axis=1, keepdims=True)


def _node1(h2, parts, w1, b1, g1, be1, w2, b2, wf_pad, bf_pad, eps):
    N = h2.shape[1]
    r = lambda v: v.reshape(1, -1)
    return pl.pallas_call(
        _node1_body,
        out_shape=jax.ShapeDtypeStruct((N, wf_pad.shape[1]), jnp.float32),
    )(h2, parts, w1, r(b1), r(g1), r(be1), w2, r(b2), wf_pad, r(bf_pad),
      eps.reshape(1, 1))


def kernel(x, edge_index, edge_attr, le0_W, le0_b, eps0, W1_0, b1_0, g1_0,
           be1_0, W2_0, b2_0, og_0, ob_0, le1_W, le1_b, eps1, W1_1, b1_1,
           g1_1, be1_1, W2_1, b2_1, og_1, ob_1, Wf, bf):
    N, D = x.shape
    E = edge_index.shape[1]
    C = Wf.shape[1]
    Dh = D // NC

    src = edge_index[0]
    dst = edge_index[1]
    EPW = E // NS
    NCF = EPW // CH
    NCH = NCF + (1 if EPW % CH else 0)
    srcw = src.reshape(NS, EPW)
    dstw = dst.reshape(NS, EPW)
    parts_s = [srcw[:, :NCF * CH].reshape(NS, NCF, CH)]
    parts_d = [dstw[:, :NCF * CH].reshape(NS, NCF, CH)]
    if EPW % CH:
        dup = NCH * CH - EPW
        parts_s.append(srcw[:, EPW - CH:].reshape(NS, 1, CH))
        dst_last = jnp.where(jnp.arange(CH) < dup, N, dstw[:, EPW - CH:])
        parts_d.append(dst_last.reshape(NS, 1, CH))
    src_m = jnp.concatenate(parts_s, axis=1) if len(parts_s) > 1 else parts_s[0]
    dst_m = jnp.concatenate(parts_d, axis=1) if len(parts_d) > 1 else parts_d[0]

    zeros_acc = jnp.zeros((N, Dh), jnp.float32)
    x2 = jnp.stack([x[:, :Dh], x[:, Dh:]])

    wf_pad = jnp.zeros((Wf.shape[0], 128), jnp.float32).at[:, :C].set(Wf)
    bf_pad = jnp.full((128,), -1e30, jnp.float32).at[:C].set(bf)

    e0 = _edge_embed(edge_attr, le0_W, le0_b)
    e1 = _edge_embed(edge_attr, le1_W, le1_b)

    parts0 = _sc_edge_stage(x2, e0, src_m, dst_m, zeros_acc)
    h2 = _node0(x, parts0, W1_0, b1_0, g1_0, be1_0, W2_0, b2_0, og_0, ob_0,
                eps0)
    parts1 = _sc_edge_stage(h2, e1, src_m, dst_m, zeros_acc)
    probs = _node1(h2, parts1, W1_1, b1_1, g1_1, be1_1, W2_1, b2_1, wf_pad,
                   bf_pad, eps1)
    return probs[:, :C]


# speedup vs baseline: 2.7879x; 1.0232x over previous
import functools

import jax
import jax.numpy as jnp
from jax import lax
from jax.experimental import pallas as pl
from jax.experimental.pallas import tpu as pltpu
from jax.experimental.pallas import tpu_sc as plsc

NC = 2
NS = 16
CH = 128


def _edge_embed_body(ea_ref, w_ref, b_ref, o_ref):
    DE = w_ref.shape[0]
    for k in range(8):
        lhs = ea_ref[:, k * DE:(k + 1) * DE]
        o_ref[:, k, :] = (
            jnp.dot(lhs, w_ref[...], preferred_element_type=jnp.float32)
            + b_ref[...]
        )


def _edge_embed(ea8, w, b, blk8=1000):
    E8 = ea8.shape[0]
    DE = w.shape[0]
    D = w.shape[1]
    return pl.pallas_call(
        _edge_embed_body,
        grid=(E8 // blk8,),
        in_specs=[
            pl.BlockSpec((blk8, 8 * DE), lambda i: (i, 0)),
            pl.BlockSpec((DE, D), lambda i: (0, 0)),
            pl.BlockSpec((1, D), lambda i: (0, 0)),
        ],
        out_specs=pl.BlockSpec((blk8, 8, D), lambda i: (i, 0, 0)),
        out_shape=jax.ShapeDtypeStruct((E8, 8, D), jnp.float32),
    )(ea8, w, b.reshape(1, D))


@functools.lru_cache(maxsize=None)
def _make_sc_edge_kernel(N, D, E, NCH):
    EPW = E // NS
    Dh = D // NC
    RPT = ((N + NS - 1) // NS + 7) // 8 * 8
    LASTR = N - RPT * (NS - 1)
    NLSL = Dh // 16
    ACC_N = N + 8

    mesh = plsc.VectorSubcoreMesh(core_axis_name="c", subcore_axis_name="s")

    @functools.partial(
        pl.kernel,
        out_type=jax.ShapeDtypeStruct((NC, N, Dh), jnp.float32),
        mesh=mesh,
        compiler_params=pltpu.CompilerParams(use_tc_tiling_on_sc=False),
        scratch_types=[
            pltpu.VMEM((NCH, CH), jnp.int32),
            pltpu.VMEM((NCH, CH), jnp.int32),
            pltpu.VMEM((CH, D // NC), jnp.float32),
            pltpu.VMEM((CH, D // NC), jnp.float32),
            pltpu.VMEM((CH // 8, 8, D // NC), jnp.float32),
            pltpu.VMEM((CH // 8, 8, D // NC), jnp.float32),
            pltpu.VMEM((CH, D // NC), jnp.float32),
            pltpu.VMEM_SHARED((ACC_N, D // NC), jnp.float32),
            pltpu.SemaphoreType.DMA,
            pltpu.SemaphoreType.DMA,
            pltpu.SemaphoreType.DMA,
            pltpu.SemaphoreType.DMA,
        ],
    )
    def k(x_hbm, e_hbm, srcm_hbm, dstm_hbm, z_hbm, out_hbm,
          sidx, didx, xb0, xb1, eb0, eb1, mbuf, acc, g0, g1, e0sem, e1sem):
        c = lax.axis_index("c")
        s = lax.axis_index("s")
        cbase = c * Dh

        @pl.when(s < NS - 1)
        def _():
            pltpu.sync_copy(z_hbm.at[pl.ds(s * RPT, RPT)],
                            acc.at[pl.ds(s * RPT, RPT)])

        @pl.when(s == NS - 1)
        def _():
            pltpu.sync_copy(z_hbm.at[pl.ds((NS - 1) * RPT, LASTR)],
                            acc.at[pl.ds((NS - 1) * RPT, LASTR)])

        pltpu.sync_copy(srcm_hbm.at[s], sidx)
        pltpu.sync_copy(dstm_hbm.at[s], didx)
        plsc.subcore_barrier()

        def start_loads(j, xb, eb, gsem, esem):
            eoff = s * EPW + jnp.minimum(j * CH, EPW - CH)
            pltpu.async_copy(x_hbm.at[c].at[sidx.at[j]], xb, gsem)
            pltpu.async_copy(
                e_hbm.at[pl.ds(lax.div(eoff, 8), CH // 8), :,
                         pl.ds(cbase, Dh)], eb, esem)

        def process(j, xb, eb, gsem, esem):
            pltpu.make_async_copy(x_hbm.at[c].at[pl.ds(0, CH)], xb, gsem).wait()
            pltpu.make_async_copy(
                e_hbm.at[pl.ds(0, CH // 8), :, pl.ds(0, Dh)], eb, esem).wait()

            @pl.loop(0, CH)
            def _row(r):
                r8 = lax.div(r, 8)
                k8 = lax.rem(r, 8)
                for t in range(NLSL):
                    sl = pl.ds(t * 16, 16)
                    mbuf[r, sl] = jnp.maximum(xb[r, sl] + eb[r8, k8, sl], 0.0)

            @pl.when(j + 2 < NCH)
            def _():
                start_loads(j + 2, xb, eb, gsem, esem)

            pltpu.sync_copy(mbuf, acc.at[didx.at[j]], add=True)

        start_loads(0, xb0, eb0, g0, e0sem)
        if NCH > 1:
            start_loads(1, xb1, eb1, g1, e1sem)

        @pl.loop(0, NCH - (NCH % 2), step=2)
        def _chunkpair(j):
            process(j, xb0, eb0, g0, e0sem)
            process(j + 1, xb1, eb1, g1, e1sem)

        if NCH % 2:
            process(NCH - 1, xb0, eb0, g0, e0sem)

        plsc.subcore_barrier()

        @pl.when(s < NS - 1)
        def _():
            pltpu.sync_copy(acc.at[pl.ds(s * RPT, RPT)],
                            out_hbm.at[c, pl.ds(s * RPT, RPT)])

        @pl.when(s == NS - 1)
        def _():
            pltpu.sync_copy(acc.at[pl.ds((NS - 1) * RPT, LASTR)],
                            out_hbm.at[c, pl.ds((NS - 1) * RPT, LASTR)])

    return k


def _sc_edge_stage(x2, e, src_m, dst_m, zeros_acc):
    _, N, Dh = x2.shape
    k = _make_sc_edge_kernel(N, Dh * NC, e.shape[0] * 8, src_m.shape[1])
    return k(x2, e, src_m, dst_m, zeros_acc)


def _bn(h, g, b):
    mu = jnp.mean(h, axis=0, keepdims=True)
    var = jnp.mean((h - mu) ** 2, axis=0, keepdims=True)
    return (h - mu) * lax.rsqrt(var + 1e-5) * g + b


def _leaky(h):
    return jnp.where(h >= 0.0, h, 0.01 * h)


def _node0_body(x_ref, p_ref, w1_ref, b1_ref, g1_ref, be1_ref, w2_ref, b2_ref,
                og_ref, ob_ref, eps_ref, o_ref):
    dh = o_ref.shape[2]
    aggr = jnp.concatenate([p_ref[0], p_ref[1]], axis=1)
    z = (1.0 + eps_ref[0, 0]) * x_ref[...] + aggr
    h = jnp.dot(z, w1_ref[...], preferred_element_type=jnp.float32) + b1_ref[...]
    h = _leaky(_bn(h, g1_ref[...], be1_ref[...]))
    h = jnp.dot(h, w2_ref[...], preferred_element_type=jnp.float32) + b2_ref[...]
    h = _leaky(_bn(h, og_ref[...], ob_ref[...]))
    o_ref[0] = h[:, :dh]
    o_ref[1] = h[:, dh:]


def _node0(x, parts, w1, b1, g1, be1, w2, b2, og, ob, eps):
    N, D = x.shape
    H = w1.shape[1]
    r = lambda v: v.reshape(1, -1)
    return pl.pallas_call(
        _node0_body,
        out_shape=jax.ShapeDtypeStruct((NC, N, H // NC), jnp.float32),
    )(x, parts, w1, r(b1), r(g1), r(be1), w2, r(b2), r(og), r(ob),
      eps.reshape(1, 1))


def _node1_body(x_ref, p_ref, w1_ref, b1_ref, g1_ref, be1_ref, w2_ref, b2_ref,
                wf_ref, bf_ref, eps_ref, o_ref):
    x = jnp.concatenate([x_ref[0], x_ref[1]], axis=1)
    aggr = jnp.concatenate([p_ref[0], p_ref[1]], axis=1)
    z = (1.0 + eps_ref[0, 0]) * x + aggr
    h = jnp.dot(z, w1_ref[...], preferred_element_type=jnp.float32) + b1_ref[...]
    h = _leaky(_bn(h, g1_ref[...], be1_ref[...]))
    h = jnp.dot(h, w2_ref[...], preferred_element_type=jnp.float32) + b2_ref[...]
    logits = (jnp.dot(h, wf_ref[...], preferred_element_type=jnp.float32)
              + bf_ref[...])
    m = jnp.max(logits, axis=1, keepdims=True)
    ex = jnp.exp(logits - m)
    o_ref[...] = ex / jnp.sum(ex, axis=1, keepdims=True)


def _node1(h2, parts, w1, b1, g1, be1, w2, b2, wf_pad, bf_pad, eps):
    N = h2.shape[1]
    r = lambda v: v.reshape(1, -1)
    return pl.pallas_call(
        _node1_body,
        out_shape=jax.ShapeDtypeStruct((N, wf_pad.shape[1]), jnp.float32),
    )(h2, parts, w1, r(b1), r(g1), r(be1), w2, r(b2), wf_pad, r(bf_pad),
      eps.reshape(1, 1))


def kernel(x, edge_index, edge_attr, le0_W, le0_b, eps0, W1_0, b1_0, g1_0,
           be1_0, W2_0, b2_0, og_0, ob_0, le1_W, le1_b, eps1, W1_1, b1_1,
           g1_1, be1_1, W2_1, b2_1, og_1, ob_1, Wf, bf):
    N, D = x.shape
    E = edge_index.shape[1]
    C = Wf.shape[1]
    Dh = D // NC

    src = edge_index[0]
    dst = edge_index[1]
    EPW = E // NS
    NCF = EPW // CH
    NCH = NCF + (1 if EPW % CH else 0)
    srcw = src.reshape(NS, EPW)
    dstw = dst.reshape(NS, EPW)
    parts_s = [srcw[:, :NCF * CH].reshape(NS, NCF, CH)]
    parts_d = [dstw[:, :NCF * CH].reshape(NS, NCF, CH)]
    if EPW % CH:
        dup = NCH * CH - EPW
        parts_s.append(srcw[:, EPW - CH:].reshape(NS, 1, CH))
        dst_last = jnp.where(jnp.arange(CH) < dup, N, dstw[:, EPW - CH:])
        parts_d.append(dst_last.reshape(NS, 1, CH))
    src_m = jnp.concatenate(parts_s, axis=1) if len(parts_s) > 1 else parts_s[0]
    dst_m = jnp.concatenate(parts_d, axis=1) if len(parts_d) > 1 else parts_d[0]

    zeros_acc = jnp.zeros((N, Dh), jnp.float32)
    x2 = jnp.stack([x[:, :Dh], x[:, Dh:]])

    wf_pad = jnp.zeros((Wf.shape[0], 128), jnp.float32).at[:, :C].set(Wf)
    bf_pad = jnp.full((128,), -1e30, jnp.float32).at[:C].set(bf)

    ea8 = edge_attr.reshape(E // 8, 8 * edge_attr.shape[1])
    e0 = _edge_embed(ea8, le0_W, le0_b)
    e1 = _edge_embed(ea8, le1_W, le1_b)

    parts0 = _sc_edge_stage(x2, e0, src_m, dst_m, zeros_acc)
    h2 = _node0(x, parts0, W1_0, b1_0, g1_0, be1_0, W2_0, b2_0, og_0, ob_0,
                eps0)
    parts1 = _sc_edge_stage(h2, e1, src_m, dst_m, zeros_acc)
    probs = _node1(h2, parts1, W1_1, b1_1, g1_1, be1_1, W2_1, b2_1, wf_pad,
                   bf_pad, eps1)
    return probs[:, :C]
